# baseline (speedup 1.0000x reference)
import functools

import numpy as np
import jax
import jax.numpy as jnp
from jax import lax
from jax.experimental import pallas as pl
from jax.experimental.pallas import tpu as pltpu
from jax.experimental.pallas import tpu_sc as plsc

_VOCAB = 1000000
_D = 64
_W = 2048
_B = 4

_NC = 2
_NS = 16
_NW = _NC * _NS
_BTOT = _B * _W
_BPW = _BTOT // _NW
_WPR = _W // _BPW
_SCALE = float(np.sqrt(_D))


def _pos_encoding() -> np.ndarray:
    half = _D // 2
    positions = np.arange(_W, dtype=np.float32)[:, None]
    depths = np.arange(half, dtype=np.float32)[None, :] / float(half)
    angle_rads = positions * (1.0 / np.power(10000.0, depths))
    return np.concatenate(
        [np.sin(angle_rads), np.cos(angle_rads)], axis=-1
    ).astype(np.float32)


_PE = _pos_encoding()


@functools.partial(
    pl.kernel,
    mesh=plsc.VectorSubcoreMesh(core_axis_name="c", subcore_axis_name="s"),
    out_type=jax.ShapeDtypeStruct((_BTOT, _D), jnp.float32),
    scratch_types=[
        pltpu.VMEM((_BPW,), jnp.int32),
        pltpu.VMEM((_BPW, _D), jnp.float32),
        pltpu.VMEM((_BPW, _D), jnp.float32),
        pltpu.SemaphoreType.DMA,
    ],
    compiler_params=pltpu.CompilerParams(use_tc_tiling_on_sc=False),
)
def _embed_pe(x_hbm, table_hbm, pe_hbm, out_hbm, idx_v, rows_v, pe_v, sem):
    wid = lax.axis_index("s") * _NC + lax.axis_index("c")
    base = wid * _BPW
    pe_base = lax.rem(wid, _WPR) * _BPW

    pltpu.sync_copy(x_hbm.at[pl.ds(base, _BPW)], idx_v)
    gather = pltpu.async_copy(table_hbm.at[idx_v], rows_v, sem)
    pltpu.sync_copy(pe_hbm.at[pl.ds(pe_base, _BPW)], pe_v)
    gather.wait()

    scale = jnp.float32(_SCALE)

    def body(i, carry):
        for r in range(4):
            row = i * 4 + r
            for j in range(_D // 16):
                sl = pl.ds(j * 16, 16)
                rows_v[row, sl] = rows_v[row, sl] * scale + pe_v[row, sl]
        return carry

    lax.fori_loop(0, _BPW // 4, body, 0, unroll=False)

    pltpu.sync_copy(rows_v, out_hbm.at[pl.ds(base, _BPW)])


def kernel(x, table):
    pe = jnp.asarray(_PE)
    idx = x.reshape(_BTOT).astype(jnp.int32)
    out = _embed_pe(idx, table, pe)
    return out.reshape(_B, _W, _D)


# speedup vs baseline: 1.6945x; 1.6945x over previous
import functools

import numpy as np
import jax
import jax.numpy as jnp
from jax import lax
from jax.experimental import pallas as pl
from jax.experimental.pallas import tpu as pltpu
from jax.experimental.pallas import tpu_sc as plsc

_VOCAB = 1000000
_D = 64
_W = 2048
_B = 4

_NC = 2
_NS = 16
_NW = _NC * _NS
_BTOT = _B * _W
_BPW = _BTOT // _NW
_WPR = _W // _BPW
_SCALE = float(np.sqrt(_D))


def _pos_encoding() -> np.ndarray:
    half = _D // 2
    positions = np.arange(_W, dtype=np.float32)[:, None]
    depths = np.arange(half, dtype=np.float32)[None, :] / float(half)
    angle_rads = positions * (1.0 / np.power(10000.0, depths))
    return np.concatenate(
        [np.sin(angle_rads), np.cos(angle_rads)], axis=-1
    ).astype(np.float32)


_PE = _pos_encoding()


@functools.partial(
    pl.kernel,
    mesh=plsc.VectorSubcoreMesh(core_axis_name="c", subcore_axis_name="s"),
    out_type=jax.ShapeDtypeStruct((_BTOT, _D), jnp.float32),
    scratch_types=[
        pltpu.VMEM((_BPW,), jnp.int32),
        pltpu.VMEM((_BPW, _D), jnp.float32),
        pltpu.VMEM((_BPW, _D), jnp.float32),
        pltpu.SemaphoreType.DMA,
        pltpu.SemaphoreType.DMA,
    ],
)
def _embed_pe(x_hbm, table_hbm, pe_hbm, out_hbm, idx_v, rows_v, pe_v, sem, gsem):
    wid = lax.axis_index("s") * _NC + lax.axis_index("c")
    base = wid * _BPW
    pe_base = lax.rem(wid, _WPR) * _BPW

    pltpu.sync_copy(x_hbm.at[pl.ds(base, _BPW)], idx_v)

    def fire(g, carry):
        vec = idx_v[pl.ds(g * 16, 16)]
        for l in range(16):
            row = vec[l]
            pltpu.async_copy(table_hbm.at[row], rows_v.at[g * 16 + l], gsem)
        return carry

    lax.fori_loop(0, _BPW // 16, fire, 0, unroll=False)

    pltpu.sync_copy(pe_hbm.at[pl.ds(pe_base, _BPW)], pe_v)

    pltpu.make_async_copy(table_hbm.at[pl.ds(0, _BPW)], rows_v, gsem).wait()

    scale = jnp.float32(_SCALE)

    def body(i, carry):
        for r in range(4):
            row = i * 4 + r
            for j in range(_D // 16):
                sl = pl.ds(j * 16, 16)
                rows_v[row, sl] = rows_v[row, sl] * scale + pe_v[row, sl]
        return carry

    lax.fori_loop(0, _BPW // 4, body, 0, unroll=False)

    pltpu.sync_copy(rows_v, out_hbm.at[pl.ds(base, _BPW)])


def kernel(x, table):
    pe = jnp.asarray(_PE)
    idx = x.reshape(_BTOT).astype(jnp.int32)
    out = _embed_pe(idx, table, pe)
    return out.reshape(_B, _W, _D)


# speedup vs baseline: 4.0493x; 2.3897x over previous
import functools

import numpy as np
import jax
import jax.numpy as jnp
from jax import lax
from jax.experimental import pallas as pl
from jax.experimental.pallas import tpu as pltpu
from jax.experimental.pallas import tpu_sc as plsc

_VOCAB = 1000000
_D = 64
_W = 2048
_B = 4

_NC = 2
_NS = 16
_NW = _NC * _NS
_BTOT = _B * _W
_BPW = _BTOT // _NW
_WPR = _W // _BPW
_SCALE = float(np.sqrt(_D))
_LANES = 128


def _pos_encoding() -> np.ndarray:
    half = _D // 2
    positions = np.arange(_W, dtype=np.float32)[:, None]
    depths = np.arange(half, dtype=np.float32)[None, :] / float(half)
    angle_rads = positions * (1.0 / np.power(10000.0, depths))
    return np.concatenate(
        [np.sin(angle_rads), np.cos(angle_rads)], axis=-1
    ).astype(np.float32)


_PE = _pos_encoding()


@functools.partial(
    pl.kernel,
    mesh=plsc.VectorSubcoreMesh(core_axis_name="c", subcore_axis_name="s"),
    out_type=jax.ShapeDtypeStruct((_BTOT, _D), jnp.float32),
    scratch_types=[
        pltpu.VMEM((_BPW + 16,), jnp.int32),
        pltpu.VMEM((_BPW, _D), jnp.float32),
        pltpu.VMEM((_BPW, _D), jnp.float32),
        pltpu.VMEM((2, _D, _LANES), jnp.float32),
        pltpu.VMEM((2, _D, _LANES), jnp.float32),
        pltpu.SemaphoreType.DMA,
        pltpu.SemaphoreType.DMA,
    ],
    compiler_params=pltpu.CompilerParams(needs_layout_passes=False),
)
def _embed_pe(x_hbm, tablet_hbm, pe_hbm, out_hbm,
              idx_v, rows_v, pe_v, tbuf_a, tbuf_b, sem_a, sem_b):
    wid = lax.axis_index("s") * _NC + lax.axis_index("c")
    base = wid * _BPW
    part = lax.rem(wid, _WPR)

    pltpu.sync_copy(x_hbm.at[pl.ds(base, _BPW)], idx_v.at[pl.ds(0, _BPW)])

    iota16 = lax.iota(jnp.int32, 16)
    scale = jnp.float32(_SCALE)

    def fire(tb, sem, fvec, lane0):
        for t in range(2):
            v = fvec[lane0 + t]
            col = pl.multiple_of(v - lax.rem(v, _LANES), _LANES)
            pltpu.async_copy(tablet_hbm.at[:, pl.ds(col, _LANES)], tb.at[t], sem)

    def wait_batch(tb, sem):
        for t in range(2):
            pltpu.make_async_copy(
                tablet_hbm.at[:, pl.ds(0, _LANES)], tb.at[t], sem
            ).wait()

    def extract(tb, svec, lane0, ibase):
        for t in range(2):
            v = svec[lane0 + t]
            lvec = jnp.full((16,), lax.rem(v, _LANES), jnp.int32)
            row = ibase + t
            for g in range(4):
                sl = pl.ds(g * 16, 16)
                vals = plsc.load_gather(tb.at[t], [iota16 + g * 16, lvec])
                rows_v[row, sl] = vals * scale + pe_v[row, sl]

    vec0 = idx_v[pl.ds(0, 16)]
    fire(tbuf_a, sem_a, vec0, 0)
    fire(tbuf_b, sem_b, vec0, 2)

    pltpu.sync_copy(pe_hbm.at[pl.ds(part * _BPW, _BPW)], pe_v)

    def body(s, carry):
        vec = idx_v[pl.ds(s * 16, 16)]
        vec_next = idx_v[pl.ds(s * 16 + 16, 16)]
        for k in range(8):
            tb, sem = (tbuf_a, sem_a) if k % 2 == 0 else (tbuf_b, sem_b)
            wait_batch(tb, sem)
            extract(tb, vec, 2 * k, s * 16 + 2 * k)
            fvec = vec if k < 6 else vec_next
            lane0 = (2 * (k + 2)) % 16

            @pl.when(8 * s + k + 2 < 128)
            def _():
                fire(tb, sem, fvec, lane0)

        return carry

    lax.fori_loop(0, 16, body, 0, unroll=False)

    pltpu.sync_copy(rows_v, out_hbm.at[pl.ds(base, _BPW)])


def kernel(x, table):
    pe = jnp.asarray(_PE)
    idx = x.reshape(_BTOT).astype(jnp.int32)
    out = _embed_pe(idx, table.T, pe)
    return out.reshape(_B, _W, _D)


# speedup vs baseline: 5.1371x; 1.2686x over previous
import functools

import numpy as np
import jax
import jax.numpy as jnp
from jax import lax
from jax.experimental import pallas as pl
from jax.experimental.pallas import tpu as pltpu
from jax.experimental.pallas import tpu_sc as plsc

_VOCAB = 1000000
_D = 64
_W = 2048
_B = 4

_NC = 2
_NS = 16
_NW = _NC * _NS
_BTOT = _B * _W
_BPW = _BTOT // _NW
_WPR = _W // _BPW
_SCALE = float(np.sqrt(_D))
_LANES = 128
_RING = 8


def _pos_encoding() -> np.ndarray:
    half = _D // 2
    positions = np.arange(_W, dtype=np.float32)[:, None]
    depths = np.arange(half, dtype=np.float32)[None, :] / float(half)
    angle_rads = positions * (1.0 / np.power(10000.0, depths))
    return np.concatenate(
        [np.sin(angle_rads), np.cos(angle_rads)], axis=-1
    ).astype(np.float32)


_PE = _pos_encoding()


@functools.partial(
    pl.kernel,
    mesh=plsc.VectorSubcoreMesh(core_axis_name="c", subcore_axis_name="s"),
    out_type=jax.ShapeDtypeStruct((_BTOT, _D), jnp.float32),
    scratch_types=[
        pltpu.VMEM((_BPW + 16,), jnp.int32),
        pltpu.VMEM((_BPW, _D), jnp.float32),
        pltpu.VMEM((_RING, _D, _LANES), jnp.float32),
        [pltpu.SemaphoreType.DMA] * _RING,
    ],
    compiler_params=pltpu.CompilerParams(needs_layout_passes=False),
)
def _embed_pe(x_hbm, tablet_hbm, pe_hbm, out_hbm, idx_v, rows_v, tbuf, sems):
    wid = lax.axis_index("s") * _NC + lax.axis_index("c")
    base = wid * _BPW
    part = lax.rem(wid, _WPR)

    pltpu.sync_copy(x_hbm.at[pl.ds(base, _BPW)], idx_v.at[pl.ds(0, _BPW)])

    iota16 = lax.iota(jnp.int32, 16)
    scale = jnp.float32(_SCALE)

    def fire(r, fvec, lane):
        v = fvec[lane]
        col = pl.multiple_of(v - lax.rem(v, _LANES), _LANES)
        pltpu.async_copy(tablet_hbm.at[:, pl.ds(col, _LANES)], tbuf.at[r], sems[r])

    vec0 = idx_v[pl.ds(0, 16)]
    for r in range(_RING):
        fire(r, vec0, r)

    pltpu.sync_copy(pe_hbm.at[pl.ds(part * _BPW, _BPW)], rows_v)

    def body(s, carry):
        vec = idx_v[pl.ds(s * 16, 16)]
        vec_next = idx_v[pl.ds(s * 16 + 16, 16)]
        for k in range(16):
            r = k % _RING
            i = s * 16 + k
            pltpu.make_async_copy(
                tablet_hbm.at[:, pl.ds(0, _LANES)], tbuf.at[r], sems[r]
            ).wait()
            v = vec[k]
            lvec = jnp.full((16,), lax.rem(v, _LANES), jnp.int32)
            for g in range(4):
                sl = pl.ds(g * 16, 16)
                vals = plsc.load_gather(tbuf.at[r], [iota16 + g * 16, lvec])
                rows_v[i, sl] = vals * scale + rows_v[i, sl]

            fvec = vec if k < 16 - _RING else vec_next
            lane = (k + _RING) % 16

            @pl.when(i + _RING < _BPW)
            def _():
                fire(r, fvec, lane)

        return carry

    lax.fori_loop(0, _BPW // 16, body, 0, unroll=False)

    pltpu.sync_copy(rows_v, out_hbm.at[pl.ds(base, _BPW)])


def kernel(x, table):
    pe = jnp.asarray(_PE)
    idx = x.reshape(_BTOT).astype(jnp.int32)
    out = _embed_pe(idx, table.T, pe)
    return out.reshape(_B, _W, _D)
